# baseline (speedup 1.0000x reference)
import functools

import jax
import jax.numpy as jnp
from jax import lax
from jax.experimental import pallas as pl
from jax.experimental.pallas import tpu as pltpu

_N = 4096
_D = 64
_BJ = 128
_NB = _N // _BJ

_NT_DIMS = (((1,), (1,)), ((), ()))


def _body(x_ref, ws_ref, wu_ref, wd_ref, lu_ref, ld_ref, out_ref,
          hs_ref, yu_ref, yd_ref):
    phase = pl.program_id(0)
    j = pl.program_id(1)

    @pl.when(phase == 0)
    def _():
        x = x_ref[...].astype(jnp.bfloat16)
        cols = pl.ds(j * _BJ, _BJ)
        hs_ref[:, cols] = jnp.tanh(
            lax.dot_general(x, ws_ref[...].astype(jnp.bfloat16), _NT_DIMS,
                            preferred_element_type=jnp.float32))
        yu_ref[:, cols] = lax.dot_general(
            x, wu_ref[...].astype(jnp.bfloat16), _NT_DIMS,
            preferred_element_type=jnp.float32).astype(jnp.bfloat16)
        yd_ref[:, cols] = lax.dot_general(
            x, wd_ref[...].astype(jnp.bfloat16), _NT_DIMS,
            preferred_element_type=jnp.float32).astype(jnp.bfloat16)

    @pl.when(phase == 1)
    def _():
        zu = jnp.dot(yu_ref[...], lu_ref[...].astype(jnp.bfloat16),
                     preferred_element_type=jnp.float32)
        zd = jnp.dot(yd_ref[...], ld_ref[...].astype(jnp.bfloat16),
                     preferred_element_type=jnp.float32)
        out_ref[...] = hs_ref[:, pl.ds(j * _BJ, _BJ)] + jnp.tanh(zu) + jnp.tanh(zd)


@functools.partial(jax.jit, static_argnames=())
def kernel(X, L_u, L_d, W_s, W_u, W_d):
    f32 = jnp.float32

    def w_idx(p, j):
        return (jnp.where(p == 0, j, _NB - 1), 0)

    def l_idx(p, j):
        return (0, jnp.where(p == 0, 0, j))

    return pl.pallas_call(
        _body,
        grid=(2, _NB),
        in_specs=[
            pl.BlockSpec((_D, _N), lambda p, j: (0, 0)),
            pl.BlockSpec((_BJ, _N), w_idx),
            pl.BlockSpec((_BJ, _N), w_idx),
            pl.BlockSpec((_BJ, _N), w_idx),
            pl.BlockSpec((_N, _BJ), l_idx),
            pl.BlockSpec((_N, _BJ), l_idx),
        ],
        out_specs=pl.BlockSpec((_D, _BJ), l_idx),
        out_shape=jax.ShapeDtypeStruct((_D, _N), f32),
        scratch_shapes=[
            pltpu.VMEM((_D, _N), f32),
            pltpu.VMEM((_D, _N), jnp.bfloat16),
            pltpu.VMEM((_D, _N), jnp.bfloat16),
        ],
        compiler_params=pltpu.CompilerParams(
            dimension_semantics=("arbitrary", "arbitrary"),
        ),
    )(X, W_s, W_u, W_d, L_u, L_d)


# speedup vs baseline: 1.0988x; 1.0988x over previous
import functools

import jax
import jax.numpy as jnp
from jax import lax
from jax.experimental import pallas as pl
from jax.experimental.pallas import tpu as pltpu

_N = 4096
_D = 64
_BJ = 256
_NB = _N // _BJ

_NT_DIMS = (((1,), (1,)), ((), ()))


def _body(x_ref, ws_ref, wu_ref, wd_ref, lu_ref, ld_ref, out_ref,
          hs_ref, yu_ref, yd_ref, zu_ref, zd_ref):
    phase = pl.program_id(0)
    j = pl.program_id(1)

    @pl.when(phase == 0)
    def _():
        x = x_ref[...].astype(jnp.bfloat16)
        cols = pl.ds(j * _BJ, _BJ)
        hs_ref[:, cols] = jnp.tanh(
            lax.dot_general(x, ws_ref[...].astype(jnp.bfloat16), _NT_DIMS,
                            preferred_element_type=jnp.float32))
        yu_ref[:, cols] = lax.dot_general(
            x, wu_ref[...].astype(jnp.bfloat16), _NT_DIMS,
            preferred_element_type=jnp.float32).astype(jnp.bfloat16)
        yd_ref[:, cols] = lax.dot_general(
            x, wd_ref[...].astype(jnp.bfloat16), _NT_DIMS,
            preferred_element_type=jnp.float32).astype(jnp.bfloat16)

    @pl.when(phase == 1)
    def _():
        ks = pl.ds(j * _BJ, _BJ)
        zu = jnp.dot(yu_ref[:, ks], lu_ref[...].astype(jnp.bfloat16),
                     preferred_element_type=jnp.float32)
        zd = jnp.dot(yd_ref[:, ks], ld_ref[...].astype(jnp.bfloat16),
                     preferred_element_type=jnp.float32)

        @pl.when(j == 0)
        def _():
            zu_ref[...] = zu
            zd_ref[...] = zd

        @pl.when(j > 0)
        def _():
            zu_ref[...] += zu
            zd_ref[...] += zd

        @pl.when(j == _NB - 1)
        def _():
            out_ref[...] = (hs_ref[...] + jnp.tanh(zu_ref[...])
                            + jnp.tanh(zd_ref[...]))


@functools.partial(jax.jit, static_argnames=())
def kernel(X, L_u, L_d, W_s, W_u, W_d):
    f32 = jnp.float32

    def w_idx(p, j):
        return (jnp.where(p == 0, j, _NB - 1), 0)

    def l_idx(p, j):
        return (jnp.where(p == 0, 0, j), 0)

    return pl.pallas_call(
        _body,
        grid=(2, _NB),
        in_specs=[
            pl.BlockSpec((_D, _N), lambda p, j: (0, 0)),
            pl.BlockSpec((_BJ, _N), w_idx),
            pl.BlockSpec((_BJ, _N), w_idx),
            pl.BlockSpec((_BJ, _N), w_idx),
            pl.BlockSpec((_BJ, _N), l_idx),
            pl.BlockSpec((_BJ, _N), l_idx),
        ],
        out_specs=pl.BlockSpec((_D, _N), lambda p, j: (0, 0)),
        out_shape=jax.ShapeDtypeStruct((_D, _N), f32),
        scratch_shapes=[
            pltpu.VMEM((_D, _N), f32),
            pltpu.VMEM((_D, _N), jnp.bfloat16),
            pltpu.VMEM((_D, _N), jnp.bfloat16),
            pltpu.VMEM((_D, _N), f32),
            pltpu.VMEM((_D, _N), f32),
        ],
        compiler_params=pltpu.CompilerParams(
            dimension_semantics=("arbitrary", "arbitrary"),
        ),
    )(X, W_s, W_u, W_d, L_u, L_d)


# speedup vs baseline: 1.1237x; 1.0227x over previous
import functools

import jax
import jax.numpy as jnp
from jax import lax
from jax.experimental import pallas as pl
from jax.experimental.pallas import tpu as pltpu

_N = 4096
_D = 64
_BJ = 256
_NB = _N // _BJ

_NT_DIMS = (((1,), (1,)), ((), ()))


def _body(x_ref, ws_ref, wu_ref, wd_ref, lu_ref, ld_ref, out_ref,
          hs_ref, yu_ref, yd_ref):
    phase = pl.program_id(0)
    j = pl.program_id(1)

    @pl.when(phase == 0)
    def _():
        x = x_ref[...].astype(jnp.bfloat16)
        cols = pl.ds(j * _BJ, _BJ)
        hs_ref[:, cols] = jnp.tanh(
            lax.dot_general(x, ws_ref[...].astype(jnp.bfloat16), _NT_DIMS,
                            preferred_element_type=jnp.float32))
        yu_ref[:, cols] = lax.dot_general(
            x, wu_ref[...].astype(jnp.bfloat16), _NT_DIMS,
            preferred_element_type=jnp.float32).astype(jnp.bfloat16)
        yd_ref[:, cols] = lax.dot_general(
            x, wd_ref[...].astype(jnp.bfloat16), _NT_DIMS,
            preferred_element_type=jnp.float32).astype(jnp.bfloat16)

    @pl.when(phase == 1)
    def _():
        zu = jnp.dot(yu_ref[...], lu_ref[...].astype(jnp.bfloat16),
                     preferred_element_type=jnp.float32)
        zd = jnp.dot(yd_ref[...], ld_ref[...].astype(jnp.bfloat16),
                     preferred_element_type=jnp.float32)
        out_ref[...] = hs_ref[:, pl.ds(j * _BJ, _BJ)] + jnp.tanh(zu) + jnp.tanh(zd)


@functools.partial(jax.jit, static_argnames=())
def kernel(X, L_u, L_d, W_s, W_u, W_d):
    f32 = jnp.float32

    def w_idx(p, j):
        return (jnp.where(p == 0, j, _NB - 1), 0)

    def l_idx(p, j):
        return (0, jnp.where(p == 0, 0, j))

    return pl.pallas_call(
        _body,
        grid=(2, _NB),
        in_specs=[
            pl.BlockSpec((_D, _N), lambda p, j: (0, 0)),
            pl.BlockSpec((_BJ, _N), w_idx),
            pl.BlockSpec((_BJ, _N), w_idx),
            pl.BlockSpec((_BJ, _N), w_idx),
            pl.BlockSpec((_N, _BJ), l_idx),
            pl.BlockSpec((_N, _BJ), l_idx),
        ],
        out_specs=pl.BlockSpec((_D, _BJ), l_idx),
        out_shape=jax.ShapeDtypeStruct((_D, _N), f32),
        scratch_shapes=[
            pltpu.VMEM((_D, _N), f32),
            pltpu.VMEM((_D, _N), jnp.bfloat16),
            pltpu.VMEM((_D, _N), jnp.bfloat16),
        ],
        compiler_params=pltpu.CompilerParams(
            dimension_semantics=("arbitrary", "arbitrary"),
        ),
    )(X, W_s, W_u, W_d, L_u, L_d)
